# Initial kernel scaffold; baseline (speedup 1.0000x reference)
import jax
import jax.numpy as jnp
from jax.experimental import pallas as pl


def kernel(herb_feature, herb_original_feat, graph1_edges, graph2_edges, p_targets, n_targets, total_map, W_lin, Wa1, ba1, Wa2, ba2, Wc1, Wc2, W_proj, b_proj, W_conf, b_conf):
    raise NotImplementedError("write your pallas kernel here")


# speedup vs baseline: 35.8493x; 35.8493x over previous
import functools

import jax
import jax.numpy as jnp
from jax import lax
from jax.experimental import pallas as pl
from jax.experimental.pallas import tpu as pltpu
from jax.experimental.pallas import tpu_sc as plsc

_B = 2
_D = 128
_NR = 3
_NC = 2
_NS = 16
_K = 128


def _pad_up(n, m):
    return ((n + m - 1) // m) * m


@functools.lru_cache(maxsize=None)
def _make_spmm(N_p, n_chunks):
    rows_pc = N_p // _NS
    mesh = plsc.VectorSubcoreMesh(core_axis_name="c", subcore_axis_name="s",
                                  num_cores=_NC, num_subcores=_NS)

    @functools.partial(
        pl.kernel,
        out_type=jax.ShapeDtypeStruct((_B, _NR, N_p, _D), jnp.float32),
        mesh=mesh,
        scratch_types=[
            pltpu.VMEM_SHARED((N_p, _D), jnp.float32),
            pltpu.VMEM((_K,), jnp.int32),
            pltpu.VMEM((_K,), jnp.int32),
            pltpu.VMEM((_K, _D), jnp.float32),
            pltpu.VMEM((8, _D), jnp.float32),
            pltpu.VMEM((8, _D), jnp.float32),
            pltpu.SemaphoreType.DMA,
        ],
    )
    def spmm(tables, srcp, dstp, out, acc, idx_s, idx_d, rows, zbuf, obuf,
             gsem):
        sid = lax.axis_index("s")
        b = lax.axis_index("c")
        row0 = sid * rows_pc
        zero16 = jnp.zeros((16,), jnp.float32)
        for rr in range(8):
            for cc in range(_D // 16):
                zbuf[rr, pl.ds(cc * 16, 16)] = zero16
        for r in range(_NR):
            def zbody(j, c):
                pltpu.sync_copy(zbuf, acc.at[pl.ds(row0 + j * 8, 8)])
                return c
            lax.fori_loop(0, rows_pc // 8, zbody, 0)
            plsc.subcore_barrier()

            def ebody(k, c):
                pltpu.sync_copy(srcp.at[b, r, sid, pl.ds(k * _K, _K)], idx_s)
                pltpu.sync_copy(dstp.at[r, sid, pl.ds(k * _K, _K)], idx_d)
                pltpu.async_copy(tables.at[idx_s], rows, gsem).wait()
                pltpu.sync_copy(rows, acc.at[idx_d], add=True)
                return c
            lax.fori_loop(0, n_chunks, ebody, 0)
            plsc.subcore_barrier()

            def obody(j, c):
                pltpu.sync_copy(acc.at[pl.ds(row0 + j * 8, 8)], obuf)
                pltpu.sync_copy(obuf, out.at[b, r, pl.ds(row0 + j * 8, 8)])
                return c
            lax.fori_loop(0, rows_pc // 8, obody, 0)
            plsc.subcore_barrier()

    return spmm


@functools.lru_cache(maxsize=None)
def _make_degrees(N_p, n_chunks):
    rows_pc = N_p // _NS
    mesh = plsc.VectorSubcoreMesh(core_axis_name="c", subcore_axis_name="s",
                                  num_cores=_NC, num_subcores=_NS)

    @functools.partial(
        pl.kernel,
        out_type=jax.ShapeDtypeStruct((2 * _NR, N_p, _D), jnp.float32),
        mesh=mesh,
        scratch_types=[
            pltpu.VMEM_SHARED((N_p, _D), jnp.float32),
            pltpu.VMEM((_K,), jnp.int32),
            pltpu.VMEM((_K, _D), jnp.float32),
            pltpu.VMEM((8, _D), jnp.float32),
            pltpu.VMEM((8, _D), jnp.float32),
        ],
    )
    def deg(ones_tbl, edges_q, out, acc, idxb, ones, zb, ob):
        sid = lax.axis_index("s")
        b = lax.axis_index("c")
        row0 = sid * rows_pc
        zero16 = jnp.zeros((16,), jnp.float32)
        for rr in range(8):
            for cc in range(_D // 16):
                zb[rr, pl.ds(cc * 16, 16)] = zero16
        pltpu.sync_copy(ones_tbl, ones)
        for j3 in range(_NR):
            q = b * _NR + j3
            def zbody(j, c):
                pltpu.sync_copy(zb, acc.at[pl.ds(row0 + j * 8, 8)])
                return c
            lax.fori_loop(0, rows_pc // 8, zbody, 0)
            plsc.subcore_barrier()

            def ebody(k, c):
                pltpu.sync_copy(edges_q.at[q, sid, pl.ds(k * _K, _K)], idxb)
                pltpu.sync_copy(ones, acc.at[idxb], add=True)
                return c
            lax.fori_loop(0, n_chunks, ebody, 0)
            plsc.subcore_barrier()

            def obody(j, c):
                pltpu.sync_copy(acc.at[pl.ds(row0 + j * 8, 8)], ob)
                pltpu.sync_copy(ob, out.at[q, pl.ds(row0 + j * 8, 8)])
                return c
            lax.fori_loop(0, rows_pc // 8, obody, 0)
            plsc.subcore_barrier()

    return deg


def _prep_edges(edges, N, N_p):
    E = edges.shape[2]
    epc = E // _NS
    epc_p = _pad_up(epc, _K)
    n_chunks = epc_p // _K
    pad = epc_p - epc
    e = edges.astype(jnp.int32).reshape(_NR, 2, _NS, epc)
    src = jnp.pad(e[:, 0], ((0, 0), (0, 0), (0, pad)))
    dst = jnp.pad(e[:, 1], ((0, 0), (0, 0), (0, pad)),
                  constant_values=N)
    roff = (jnp.arange(_NR, dtype=jnp.int32) * N_p)[None, :, None, None]
    boff = (jnp.arange(_B, dtype=jnp.int32) * (_NR * N_p))[:, None, None, None]
    srcp = src[None] + roff + boff
    src_q = jnp.pad(e[:, 0], ((0, 0), (0, 0), (0, pad)), constant_values=N)
    edges_q = jnp.stack([src_q, dst], axis=1).reshape(2 * _NR, _NS, epc_p)
    return srcp, dst, edges_q, n_chunks


def _degree_scales(edges_q, N, N_p, n_chunks):
    ones_tbl = jnp.ones((_K, _D), jnp.float32)
    cnt = _make_degrees(N_p, n_chunks)(ones_tbl, edges_q)
    cnt = cnt[:, :, 0]
    sc = jax.lax.rsqrt(jnp.maximum(cnt, 1.0))
    sc = sc * (jnp.arange(N_p) < N)[None, :]
    do = sc[0::2]
    di = sc[1::2]
    return do, di


def _conv(x, W3, aw, do, di, srcp, dstp, N_p, n_chunks):
    y = jnp.einsum('bnd,rde->brne', x, W3,
                   preferred_element_type=jnp.float32)
    y = y * do[None, :, :, None]
    tables = y.reshape(_B * _NR * N_p, _D)
    agg = _make_spmm(N_p, n_chunks)(tables, srcp, dstp)
    scale = di[None, :, :, None] * aw[:, :, None, None]
    return jax.nn.relu(agg * scale).sum(axis=1)


def kernel(herb_feature, herb_original_feat, graph1_edges, graph2_edges,
           p_targets, n_targets, total_map, W_lin, Wa1, ba1, Wa2, ba2,
           Wc1, Wc2, W_proj, b_proj, W_conf, b_conf):
    B, NH = herb_original_feat.shape
    N1 = total_map.shape[1]
    N2 = int(graph2_edges.shape[2] // 16)
    LI_LO = Wa1.shape[0]
    LO = Wa2.shape[0]
    LI = LI_LO // LO
    N1p = _pad_up(N1, _NS * 8)
    N2p = _pad_up(N2, _NS * 8)

    src1, dst1, eq1, nc1 = _prep_edges(graph1_edges, N1, N1p)
    src2, dst2, eq2, nc2 = _prep_edges(graph2_edges, N2, N2p)
    do1, di1 = _degree_scales(eq1, N1, N1p, nc1)
    do2, di2 = _degree_scales(eq2, N2, N2p, nc2)

    mask = (herb_original_feat > 0).astype(jnp.float32)
    s = mask @ total_map
    s = jnp.pad(s, ((0, 0), (0, N1p - N1)))
    i_feat = s[:, :, None] * herb_feature[:, None, :]

    aw1s = jax.nn.sigmoid(jnp.einsum('bd,ldr->lbr', herb_feature, Wa1)
                          + ba1[:, None, :])
    aw2s = jax.nn.sigmoid(jnp.einsum('bd,ldr->lbr', herb_feature, Wa2)
                          + ba2[:, None, :])

    ctop = i_feat @ W_lin
    c_feat = jnp.pad(ctop, ((0, 0), (0, N2p - N1p), (0, 0)))

    for o in range(LO):
        for i in range(LI):
            idx = o * LI + i
            i_feat = _conv(i_feat, Wc1[idx], aw1s[idx], do1, di1,
                           src1, dst1, N1p, nc1)
        c_feat = _conv(c_feat, Wc2[o], aw2s[o], do2, di2, src2, dst2, N2p, nc2)
        c_feat = _conv(c_feat, Wc2[o], aw2s[o], do2, di2, src2, dst2, N2p, nc2)
        out_c = c_feat[:, :N1p, :]
        proj = out_c @ W_proj + b_proj
        conf = jax.nn.sigmoid(out_c @ W_conf + b_conf)
        f = conf * (i_feat + proj)
        i_feat = f
        if o + 1 < LO:
            c_feat = jnp.pad(f @ W_lin, ((0, 0), (0, N2p - N1p), (0, 0)))

    bidx = jnp.arange(B)
    p_feat = f[bidx, p_targets.reshape(-1), :]
    n_feat = f[bidx, n_targets.reshape(-1), :]
    return (p_feat, n_feat)
